# baseline (speedup 1.0000x reference)
import math

import jax
import jax.numpy as jnp
from jax import lax
from jax.experimental import pallas as pl
from jax.experimental.pallas import tpu as pltpu

_M = 0.99
_ONE_MINUS_M = 1.0 - _M
_LOG_M = math.log(_M)

_NUM_CLASS = 100000
_DIM = 64
_BATCH = 1024
_R = 4096
_W = 128
_NB = 32
_SLOTS = _NB * _W
_SHIFT = 12


def _tail(contrib, cnt, proto, feat, out_ref):
    decay = jnp.exp(cnt * _LOG_M)
    upd = decay * proto + _ONE_MINUS_M * contrib
    norm = jnp.sqrt(jnp.sum(upd * upd, axis=1, keepdims=True))
    upd_n = upd * (1.0 / jnp.maximum(norm, 1e-12))
    out_ref[...] = lax.dot_general(feat, upd_n,
                                   dimension_numbers=(((1,), (1,)), ((), ())),
                                   preferred_element_type=jnp.float32)


def _body(lab_col_ref, lab_row_ref, pred_ref, feat_ref, proto_ref, out_ref,
          win_ref, rl_ref, ovf_ref):
    pid = pl.program_id(0)

    @pl.when(pid == 0)
    def _init():
        lc = lab_col_ref[...]
        lr = lab_row_ref[...]
        col = lax.broadcasted_iota(jnp.int32, (_BATCH, _BATCH), 1)
        row = lax.broadcasted_iota(jnp.int32, (_BATCH, _BATCH), 0)
        eq = lc == lr
        eqb = (lc >> _SHIFT) == (lr >> _SHIFT)
        m_later = jnp.where(eq & (col > row), 1.0, 0.0).astype(jnp.bfloat16)
        m_early = jnp.where(eqb & (col < row), 1.0, 0.0).astype(jnp.bfloat16)
        ones_v = jnp.full((_BATCH, 1), 1.0, jnp.bfloat16)
        both = jnp.dot(jnp.concatenate([m_later, m_early], axis=0), ones_v,
                       preferred_element_type=jnp.float32)
        c = both[:_BATCH]
        w = jnp.exp(c * _LOG_M)
        rank = both[_BATCH:].astype(jnp.int32)
        slotkey = (lc >> _SHIFT) * _W + rank
        skey_row = jnp.transpose(slotkey, (1, 0)).astype(jnp.int16)
        rloc = (lc & (_R - 1)).astype(jnp.float32)
        ci = lax.broadcasted_iota(jnp.int32, (_BATCH, _DIM), 1)
        extras = (jnp.where(ci == 0, 1.0, 0.0)
                  + jnp.where(ci == 1, jnp.floor(rloc / 64.0), 0.0)
                  + jnp.where(ci == 2, rloc - jnp.floor(rloc / 64.0) * 64.0,
                              0.0))
        wf = jnp.concatenate([w * pred_ref[...], extras],
                             axis=1).astype(jnp.bfloat16)
        sid = lax.broadcasted_iota(jnp.int16, (_SLOTS, _BATCH), 0)
        lch = jnp.where(sid == skey_row, jnp.bfloat16(1.0), jnp.bfloat16(0.0))
        winf = jnp.dot(lch, wf, preferred_element_type=jnp.float32)
        win_ref[...] = winf.astype(jnp.bfloat16)
        rl_c = winf[:, _DIM + 1:_DIM + 2] * 64.0 + winf[:, _DIM + 2:_DIM + 3]
        rl_ref[...] = jnp.transpose(rl_c, (1, 0)).astype(jnp.int16)
        ovf_ref[0] = jnp.sum(jnp.where(rank >= _W, 1, 0))

    overflow = ovf_ref[0] > 0

    @pl.when(jnp.logical_not(overflow))
    def _fast():
        win = win_ref[pl.ds(pid * _W, _W), :]
        rl_row = rl_ref[:, pl.ds(pid * _W, _W)]
        rowid = lax.broadcasted_iota(jnp.int16, (_R, _W), 0)
        st = jnp.where(rowid == rl_row, jnp.bfloat16(1.0), jnp.bfloat16(0.0))
        full = jnp.dot(st, win, preferred_element_type=jnp.float32)
        _tail(full[:, :_DIM], full[:, _DIM:_DIM + 1], proto_ref[...],
              feat_ref[...], out_ref)

    @pl.when(overflow)
    def _slow():
        lc = lab_col_ref[...]
        lr = lab_row_ref[...]
        col = lax.broadcasted_iota(jnp.int32, (_BATCH, _BATCH), 1)
        row = lax.broadcasted_iota(jnp.int32, (_BATCH, _BATCH), 0)
        eq = lc == lr
        c = jnp.sum(jnp.where(eq & (col > row), 1.0, 0.0), axis=1,
                    keepdims=True)
        w = jnp.exp(c * _LOG_M)
        ci = lax.broadcasted_iota(jnp.int32, (_BATCH, _DIM), 1)
        ones_col = jnp.where(ci == 0, 1.0, 0.0)
        wfull = jnp.concatenate([w * pred_ref[...], ones_col],
                                axis=1).astype(jnp.bfloat16)
        rel = jnp.clip(lr - pid * _R, -1, _R).astype(jnp.int16)
        rowid = lax.broadcasted_iota(jnp.int16, (_R, _BATCH), 0)
        st = jnp.where(rowid == rel, jnp.bfloat16(1.0), jnp.bfloat16(0.0))
        full = jnp.dot(st, wfull, preferred_element_type=jnp.float32)
        _tail(full[:, :_DIM], full[:, _DIM:_DIM + 1], proto_ref[...],
              feat_ref[...], out_ref)


@jax.jit
def kernel(pred_feat, pseudo_label, feat, prototypes):
    lab = pseudo_label.astype(jnp.int32)
    lab_col = lab.reshape(_BATCH, 1)
    lab_row = lab.reshape(1, _BATCH)
    grid = (pl.cdiv(_NUM_CLASS, _R),)
    return pl.pallas_call(
        _body,
        grid=grid,
        in_specs=[
            pl.BlockSpec((_BATCH, 1), lambda i: (0, 0)),
            pl.BlockSpec((1, _BATCH), lambda i: (0, 0)),
            pl.BlockSpec((_BATCH, _DIM), lambda i: (0, 0)),
            pl.BlockSpec((_BATCH, _DIM), lambda i: (0, 0)),
            pl.BlockSpec((_R, _DIM), lambda i: (i, 0)),
        ],
        out_specs=pl.BlockSpec((_BATCH, _R), lambda i: (0, i)),
        out_shape=jax.ShapeDtypeStruct((_BATCH, _NUM_CLASS), jnp.float32),
        scratch_shapes=[
            pltpu.VMEM((_SLOTS, 2 * _DIM), jnp.bfloat16),
            pltpu.VMEM((1, _SLOTS), jnp.int16),
            pltpu.SMEM((1,), jnp.int32),
        ],
        compiler_params=pltpu.CompilerParams(
            dimension_semantics=("arbitrary",)),
    )(lab_col, lab_row, pred_feat, feat, prototypes)
